# baseline (speedup 1.0000x reference)
import jax
import jax.numpy as jnp
import numpy as np
from jax import lax
from jax.experimental import pallas as pl
from jax.experimental.pallas import tpu as pltpu

N_FRAMES = 240
HEIGHT = 32
WIDTH = 32
N_DIMS = 64
KSIZE = 5
PAD = KSIZE // 2
TEMP = 5.0
MAX_NORM = float(N_DIMS)
ROW = HEIGHT * WIDTH * N_DIMS
NROWS = N_FRAMES + 2 * PAD
NPAD = 256
B = 256
RB = 32
NA = (NROWS + RB - 1) // RB
CC = 2048
NC = ROW // CC

_W = np.exp(-((np.arange(KSIZE, dtype=np.float32) - PAD) ** 2) / np.float32(TEMP))
_W = (_W / _W.sum()).astype(np.float32)


def _fused_body(idx_ref, x_ref, out_ref, tb_ref, acc_ref, w_ref):
    i = pl.program_id(0)

    @pl.when(i < NA)
    def _():
        x = x_ref[...]
        rows = lax.broadcasted_iota(jnp.int32, (RB, 1), 0) + i * RB
        x = jnp.where(rows < NROWS, x, 0.0)
        gram = lax.dot_general(
            x, x, dimension_numbers=(((1,), (1,)), ((), ())),
            preferred_element_type=jnp.float32,
        )
        eye = (
            lax.broadcasted_iota(jnp.int32, (RB, RB), 0)
            == lax.broadcasted_iota(jnp.int32, (RB, RB), 1)
        )
        ss = jnp.sum(jnp.where(eye, gram, 0.0), axis=1, keepdims=True)
        acc_ref[pl.ds(i * RB, RB)] = jnp.broadcast_to(ss, (RB, 128))
        xb = x.astype(jnp.bfloat16).reshape(RB, NC, CC)
        for jj in range(NC):
            tb_ref[jj, pl.ds(i * RB, RB), :] = xb[:, jj, :]

    @pl.when(i == NA)
    def _():
        norm = jnp.sqrt(acc_ref[0:NPAD, 0:1])
        scale = jnp.minimum(jnp.float32(1.0), MAX_NORM / (norm + 1e-7))
        r = lax.broadcasted_iota(jnp.int32, (NPAD, B), 0)
        delta = r - idx_ref[0][None, :]
        wv = jnp.zeros((NPAD, B), jnp.float32)
        for k in range(KSIZE):
            wv = jnp.where(delta == k, _W[k], wv)
        wv = jnp.where(r < NROWS, wv * scale, 0.0)
        w_ref[...] = wv.astype(jnp.bfloat16)

    @pl.when(i > NA)
    def _():
        j = i - NA - 1
        lhs = tb_ref[j]
        out_ref[...] = lax.dot_general(
            lhs, w_ref[...],
            dimension_numbers=(((0,), (0,)), ((), ())),
            preferred_element_type=jnp.float32,
        )


def _fused(idxs2d, table):
    return pl.pallas_call(
        _fused_body,
        grid=(NA + 1 + NC,),
        in_specs=[
            pl.BlockSpec((1, B), lambda i: (0, 0)),
            pl.BlockSpec((RB, ROW), lambda i: (jnp.minimum(i, NA - 1), 0)),
        ],
        out_specs=pl.BlockSpec(
            (CC, B), lambda i: (jnp.maximum(i - NA - 1, 0), 0)
        ),
        out_shape=jax.ShapeDtypeStruct((ROW, B), jnp.float32),
        scratch_shapes=[
            pltpu.VMEM((NC, NPAD, CC), jnp.bfloat16),
            pltpu.VMEM((NPAD, 128), jnp.float32),
            pltpu.VMEM((NPAD, B), jnp.bfloat16),
        ],
    )(idxs2d, table)


def kernel(idxs, frame_embs):
    out_cb = _fused(idxs.astype(jnp.int32).reshape(1, B), frame_embs)
    return jnp.transpose(
        out_cb.reshape(N_DIMS, HEIGHT, WIDTH, B), (3, 0, 1, 2)
    )


# speedup vs baseline: 1.6300x; 1.6300x over previous
import jax
import jax.numpy as jnp
import numpy as np
from jax import lax
from jax.experimental import pallas as pl
from jax.experimental.pallas import tpu as pltpu

N_FRAMES = 240
HEIGHT = 32
WIDTH = 32
N_DIMS = 64
KSIZE = 5
PAD = KSIZE // 2
TEMP = 5.0
MAX_NORM = float(N_DIMS)
ROW = HEIGHT * WIDTH * N_DIMS
NROWS = N_FRAMES + 2 * PAD
NPAD = 256
B = 256
RB = 32
NA = (NROWS + RB - 1) // RB
CC = 2048
NC = ROW // CC

_W = np.exp(-((np.arange(KSIZE, dtype=np.float32) - PAD) ** 2) / np.float32(TEMP))
_W = (_W / _W.sum()).astype(np.float32)


def _fused_body(idx_ref, x_ref, out_ref, tb_ref, acc_ref, w_ref):
    i = pl.program_id(0)

    @pl.when(i < NA)
    def _():
        x = x_ref[...]
        rows = lax.broadcasted_iota(jnp.int32, (RB, 1), 0) + i * RB
        x = jnp.where(rows < NROWS, x, 0.0)
        gram = lax.dot_general(
            x, x, dimension_numbers=(((1,), (1,)), ((), ())),
            preferred_element_type=jnp.float32,
        )
        eye = (
            lax.broadcasted_iota(jnp.int32, (RB, RB), 0)
            == lax.broadcasted_iota(jnp.int32, (RB, RB), 1)
        )
        ss = jnp.sum(jnp.where(eye, gram, 0.0), axis=1, keepdims=True)
        acc_ref[pl.ds(i * RB, RB)] = jnp.broadcast_to(ss, (RB, 128))
        tb_ref[pl.ds(i * RB, RB), :] = x.astype(jnp.bfloat16)

    @pl.when(i == NA)
    def _():
        norm = jnp.sqrt(acc_ref[0:NPAD, 0:1])
        scale = jnp.minimum(jnp.float32(1.0), MAX_NORM / (norm + 1e-7))
        r = lax.broadcasted_iota(jnp.int32, (NPAD, B), 0)
        delta = r - idx_ref[0][None, :]
        wv = jnp.zeros((NPAD, B), jnp.float32)
        for k in range(KSIZE):
            wv = jnp.where(delta == k, _W[k], wv)
        wv = jnp.where(r < NROWS, wv * scale, 0.0)
        w_ref[...] = wv.astype(jnp.bfloat16)

    @pl.when(i > NA)
    def _():
        j = i - NA - 1
        lhs = tb_ref[:, pl.ds(pl.multiple_of(j * CC, CC), CC)]
        out_ref[...] = lax.dot_general(
            lhs, w_ref[...],
            dimension_numbers=(((0,), (0,)), ((), ())),
            preferred_element_type=jnp.float32,
        )


def _fused(idxs2d, table):
    return pl.pallas_call(
        _fused_body,
        grid=(NA + 1 + NC,),
        in_specs=[
            pl.BlockSpec((1, B), lambda i: (0, 0)),
            pl.BlockSpec((RB, ROW), lambda i: (jnp.minimum(i, NA - 1), 0)),
        ],
        out_specs=pl.BlockSpec(
            (CC, B), lambda i: (jnp.maximum(i - NA - 1, 0), 0)
        ),
        out_shape=jax.ShapeDtypeStruct((ROW, B), jnp.float32),
        scratch_shapes=[
            pltpu.VMEM((NPAD, ROW), jnp.bfloat16),
            pltpu.VMEM((NPAD, 128), jnp.float32),
            pltpu.VMEM((NPAD, B), jnp.bfloat16),
        ],
    )(idxs2d, table)


def kernel(idxs, frame_embs):
    out_cb = _fused(idxs.astype(jnp.int32).reshape(1, B), frame_embs)
    return jnp.transpose(
        out_cb.reshape(N_DIMS, HEIGHT, WIDTH, B), (3, 0, 1, 2)
    )


# speedup vs baseline: 1.8900x; 1.1595x over previous
import jax
import jax.numpy as jnp
import numpy as np
from jax import lax
from jax.experimental import pallas as pl
from jax.experimental.pallas import tpu as pltpu

N_FRAMES = 240
HEIGHT = 32
WIDTH = 32
N_DIMS = 64
KSIZE = 5
PAD = KSIZE // 2
TEMP = 5.0
MAX_NORM = float(N_DIMS)
ROW = HEIGHT * WIDTH * N_DIMS
NROWS = N_FRAMES + 2 * PAD
NPAD = 256
B = 256
RB = 32
NA = (NROWS + RB - 1) // RB
CC = 4096
NC = ROW // CC

_W = np.exp(-((np.arange(KSIZE, dtype=np.float32) - PAD) ** 2) / np.float32(TEMP))
_W = (_W / _W.sum()).astype(np.float32)


def _fused_body(idx_ref, x_ref, out_ref, tb_ref, acc_ref, w_ref):
    i = pl.program_id(0)

    @pl.when(i < NA)
    def _():
        x = x_ref[...]
        rows = lax.broadcasted_iota(jnp.int32, (RB, 1), 0) + i * RB
        x = jnp.where(rows < NROWS, x, 0.0)
        gram = lax.dot_general(
            x, x, dimension_numbers=(((1,), (1,)), ((), ())),
            preferred_element_type=jnp.float32,
        )
        eye = (
            lax.broadcasted_iota(jnp.int32, (RB, RB), 0)
            == lax.broadcasted_iota(jnp.int32, (RB, RB), 1)
        )
        ss = jnp.sum(jnp.where(eye, gram, 0.0), axis=1, keepdims=True)
        acc_ref[pl.ds(i * RB, RB)] = jnp.broadcast_to(ss, (RB, 128))
        tb_ref[pl.ds(i * RB, RB), :] = x.astype(jnp.bfloat16)

    @pl.when(i == NA)
    def _():
        norm = jnp.sqrt(acc_ref[0:NPAD, 0:1])
        scale = jnp.minimum(jnp.float32(1.0), MAX_NORM / (norm + 1e-7))
        r = lax.broadcasted_iota(jnp.int32, (NPAD, B), 0)
        delta = r - idx_ref[0][None, :]
        wv = jnp.zeros((NPAD, B), jnp.float32)
        for k in range(KSIZE):
            wv = jnp.where(delta == k, _W[k], wv)
        wv = jnp.where(r < NROWS, wv * scale, 0.0)
        w_ref[...] = wv.astype(jnp.bfloat16)

    @pl.when(i > NA)
    def _():
        j = i - NA - 1
        lhs = tb_ref[:, pl.ds(pl.multiple_of(j * CC, CC), CC)]
        out_ref[...] = lax.dot_general(
            lhs, w_ref[...],
            dimension_numbers=(((0,), (0,)), ((), ())),
            preferred_element_type=jnp.float32,
        )


def _fused(idxs2d, table):
    return pl.pallas_call(
        _fused_body,
        grid=(NA + 1 + NC,),
        in_specs=[
            pl.BlockSpec((1, B), lambda i: (0, 0)),
            pl.BlockSpec((RB, ROW), lambda i: (jnp.minimum(i, NA - 1), 0)),
        ],
        out_specs=pl.BlockSpec(
            (CC, B), lambda i: (jnp.maximum(i - NA - 1, 0), 0)
        ),
        out_shape=jax.ShapeDtypeStruct((ROW, B), jnp.float32),
        scratch_shapes=[
            pltpu.VMEM((NPAD, ROW), jnp.bfloat16),
            pltpu.VMEM((NPAD, 128), jnp.float32),
            pltpu.VMEM((NPAD, B), jnp.bfloat16),
        ],
    )(idxs2d, table)


def kernel(idxs, frame_embs):
    out_cb = _fused(idxs.astype(jnp.int32).reshape(1, B), frame_embs)
    return jnp.transpose(
        out_cb.reshape(N_DIMS, HEIGHT, WIDTH, B), (3, 0, 1, 2)
    )
